# Initial kernel scaffold; baseline (speedup 1.0000x reference)
import jax
import jax.numpy as jnp
from jax.experimental import pallas as pl


def kernel(x, edge_index, train_edge_id, fc1_w, fc1_b, eps, w1, b1, w2, b2, bn_g, bn_b, lin1_w, lin1_b, lin2_w, lin2_b, fc2_w, fc2_b):
    raise NotImplementedError("write your pallas kernel here")


# speedup vs baseline: 5.8231x; 5.8231x over previous
import functools

import jax
import jax.numpy as jnp
from jax import lax
from jax.experimental import pallas as pl
from jax.experimental.pallas import tpu as pltpu
from jax.experimental.pallas import tpu_sc as plsc

_NC = 2
_NS = 16
_NW = _NC * _NS


def _fc1_body(x_ref, w_ref, b_ref, o_ref):
    o_ref[...] = (
        lax.dot_general(x_ref[...], w_ref[...], (((1,), (1,)), ((), ())),
                        preferred_element_type=jnp.float32)
        + b_ref[...]
    )


def _fc1(x, w, b):
    n, d = x.shape
    blk = 2000
    return pl.pallas_call(
        _fc1_body,
        grid=(n // blk,),
        in_specs=[
            pl.BlockSpec((blk, d), lambda i: (i, 0)),
            pl.BlockSpec((d, d), lambda i: (0, 0)),
            pl.BlockSpec((1, d), lambda i: (0, 0)),
        ],
        out_specs=pl.BlockSpec((blk, d), lambda i: (i, 0)),
        out_shape=jax.ShapeDtypeStruct((n, d), jnp.float32),
    )(x, w, b.reshape(1, d))


def _make_seg_sum(n, d, e):
    ept = e // _NW
    k = 200
    assert ept % k == 0
    rows_pt = (n // _NS) // 8 * 8
    tail = n - rows_pt * _NS

    mesh = plsc.VectorSubcoreMesh(core_axis_name="c", subcore_axis_name="s")

    @functools.partial(
        pl.kernel,
        out_type=jax.ShapeDtypeStruct((_NC, n, d), jnp.float32),
        mesh=mesh,
        scratch_types=[
            pltpu.VMEM((k,), jnp.int32),
            pltpu.VMEM((k,), jnp.int32),
            pltpu.VMEM((k, d), jnp.float32),
            pltpu.VMEM_SHARED((n, d), jnp.float32),
            pltpu.SemaphoreType.DMA,
        ],
    )
    def seg_sum(h_hbm, src_hbm, dst_hbm, zeros_hbm, out_hbm,
                src_v, dst_v, rows_v, acc_sh, sem):
        cid = lax.axis_index("c")
        sid = lax.axis_index("s")
        wid = sid * _NC + cid
        pltpu.sync_copy(zeros_hbm.at[pl.ds(sid * rows_pt, rows_pt)],
                        acc_sh.at[pl.ds(sid * rows_pt, rows_pt)])
        if tail:
            @pl.when(sid == _NS - 1)
            def _():
                pltpu.sync_copy(zeros_hbm.at[pl.ds(rows_pt * _NS, tail)],
                                acc_sh.at[pl.ds(rows_pt * _NS, tail)])
        plsc.subcore_barrier()

        base0 = wid * ept

        def body(i, carry):
            base = base0 + i * k
            pltpu.sync_copy(src_hbm.at[pl.ds(base, k)], src_v)
            pltpu.sync_copy(dst_hbm.at[pl.ds(base, k)], dst_v)
            pltpu.async_copy(h_hbm.at[src_v], rows_v, sem).wait()
            pltpu.sync_copy(rows_v, acc_sh.at[dst_v], add=True)
            return carry

        lax.fori_loop(0, ept // k, body, 0)
        plsc.subcore_barrier()
        pltpu.sync_copy(acc_sh.at[pl.ds(sid * rows_pt, rows_pt)],
                        out_hbm.at[cid, pl.ds(sid * rows_pt, rows_pt)])
        if tail:
            @pl.when(sid == _NS - 1)
            def _():
                pltpu.sync_copy(acc_sh.at[pl.ds(rows_pt * _NS, tail)],
                                out_hbm.at[cid, pl.ds(rows_pt * _NS, tail)])

    return seg_sum


def _mlp_body(scale_ref, h1_ref, a0_ref, a1_ref,
              w1_ref, b1_ref, w2_ref, b2_ref, bng_ref, bnb_ref,
              l1w_ref, l1b_ref, l2w_ref, l2b_ref, o_ref):
    u = scale_ref[0, 0] * h1_ref[...] + a0_ref[...] + a1_ref[...]
    dn = (((1,), (1,)), ((), ()))
    t = jnp.maximum(
        lax.dot_general(u, w1_ref[...], dn, preferred_element_type=jnp.float32)
        + b1_ref[...], 0.0)
    t = jnp.maximum(
        lax.dot_general(t, w2_ref[...], dn, preferred_element_type=jnp.float32)
        + b2_ref[...], 0.0)
    t = t * (bng_ref[...] * (1.0 / jnp.sqrt(1.0 + 1e-5))) + bnb_ref[...]
    t = jnp.maximum(
        lax.dot_general(t, l1w_ref[...], dn, preferred_element_type=jnp.float32)
        + l1b_ref[...], 0.0)
    o_ref[...] = (
        lax.dot_general(t, l2w_ref[...], dn, preferred_element_type=jnp.float32)
        + l2b_ref[...])


def _mlp(scale, h1, a0, a1, w1, b1, w2, b2, bn_g, bn_b, l1w, l1b, l2w, l2b):
    n, d = h1.shape
    h = w1.shape[0]
    blk = 2000
    full = lambda shape: pl.BlockSpec(shape, lambda i: tuple(0 for _ in shape))
    row = lambda width: pl.BlockSpec((blk, width), lambda i: (i, 0))
    return pl.pallas_call(
        _mlp_body,
        grid=(n // blk,),
        in_specs=[
            pl.BlockSpec(memory_space=pltpu.SMEM),
            row(d), row(d), row(d),
            full((h, d)), full((1, h)), full((h, h)), full((1, h)),
            full((1, h)), full((1, h)),
            full((h, h)), full((1, h)), full((h, h)), full((1, h)),
        ],
        out_specs=row(h),
        out_shape=jax.ShapeDtypeStruct((n, h), jnp.float32),
    )(scale, h1, a0, a1, w1, b1.reshape(1, h), w2, b2.reshape(1, h),
      bn_g.reshape(1, h), bn_b.reshape(1, h),
      l1w, l1b.reshape(1, h), l2w, l2b.reshape(1, h))


def _make_pair_gather(n, h, e, b):
    bpt = b // _NW
    m = 128
    nsub = bpt // m
    assert bpt % m == 0
    nv = h // 16

    mesh = plsc.VectorSubcoreMesh(core_axis_name="c", subcore_axis_name="s")

    @functools.partial(
        pl.kernel,
        out_type=jax.ShapeDtypeStruct((b, h), jnp.float32),
        mesh=mesh,
        scratch_types=[
            pltpu.VMEM((nsub, m), jnp.int32),
            pltpu.VMEM((nsub, m), jnp.int32),
            pltpu.VMEM((nsub, m), jnp.int32),
            pltpu.VMEM((m, h), jnp.float32),
            pltpu.VMEM((m, h), jnp.float32),
            pltpu.SemaphoreType.DMA,
            pltpu.SemaphoreType.DMA,
        ],
    )
    def pair(h_hbm, ei0_hbm, ei1_hbm, te_hbm, out_hbm,
             te_v, s_v, d_v, x1_v, x2_v, sem1, sem2):
        cid = lax.axis_index("c")
        sid = lax.axis_index("s")
        wid = sid * _NC + cid
        base = wid * bpt
        for j in range(nsub):
            pltpu.sync_copy(te_hbm.at[pl.ds(base + j * m, m)], te_v.at[j])
        for j in range(nsub):
            c1 = pltpu.async_copy(ei0_hbm.at[te_v.at[j]], s_v.at[j], sem1)
            c2 = pltpu.async_copy(ei1_hbm.at[te_v.at[j]], d_v.at[j], sem2)
            c1.wait()
            c2.wait()
        for j in range(nsub):
            c1 = pltpu.async_copy(h_hbm.at[s_v.at[j]], x1_v, sem1)
            c2 = pltpu.async_copy(h_hbm.at[d_v.at[j]], x2_v, sem2)
            c1.wait()
            c2.wait()

            def mul_row(r, carry):
                for c in range(nv):
                    sl = pl.ds(c * 16, 16)
                    x1_v[r, sl] = x1_v[r, sl] * x2_v[r, sl]
                return carry

            lax.fori_loop(0, m, mul_row, 0)
            pltpu.sync_copy(x1_v, out_hbm.at[pl.ds(base + j * m, m)])

    return pair


def _fc2_body(f_ref, w_ref, b_ref, o_ref):
    o_ref[...] = (
        lax.dot_general(f_ref[...], w_ref[...], (((1,), (1,)), ((), ())),
                        preferred_element_type=jnp.float32)
        + b_ref[...])


def _fc2(fused, w, bias):
    b, h = fused.shape
    c = w.shape[0]
    blk = 4096
    return pl.pallas_call(
        _fc2_body,
        grid=(b // blk,),
        in_specs=[
            pl.BlockSpec((blk, h), lambda i: (i, 0)),
            pl.BlockSpec((c, h), lambda i: (0, 0)),
            pl.BlockSpec((1, c), lambda i: (0, 0)),
        ],
        out_specs=pl.BlockSpec((blk, c), lambda i: (i, 0)),
        out_shape=jax.ShapeDtypeStruct((b, c), jnp.float32),
    )(fused, w, bias.reshape(1, c))


def kernel(x, edge_index, train_edge_id, fc1_w, fc1_b, eps, w1, b1, w2, b2,
           bn_g, bn_b, lin1_w, lin1_b, lin2_w, lin2_b, fc2_w, fc2_b):
    n, d = x.shape
    e = edge_index.shape[1]
    b = train_edge_id.shape[0]
    h = w1.shape[0]

    src = edge_index[0]
    dst = edge_index[1]

    h1 = _fc1(x, fc1_w, fc1_b)

    zeros = jnp.zeros((n, d), jnp.float32)
    aggs = _make_seg_sum(n, d, e)(h1, src, dst, zeros)

    scale = (1.0 + eps).reshape(1, 1)
    h5 = _mlp(scale, h1, aggs[0], aggs[1], w1, b1, w2, b2, bn_g, bn_b,
              lin1_w, lin1_b, lin2_w, lin2_b)

    fused = _make_pair_gather(n, h, e, b)(h5, src, dst, train_edge_id)

    return _fc2(fused, fc2_w, fc2_b)


# speedup vs baseline: 7.3995x; 1.2707x over previous
import functools

import jax
import jax.numpy as jnp
from jax import lax
from jax.experimental import pallas as pl
from jax.experimental.pallas import tpu as pltpu
from jax.experimental.pallas import tpu_sc as plsc

_NC = 2
_NS = 16
_NW = _NC * _NS


def _fc1_body(x_ref, w_ref, b_ref, o_ref):
    o_ref[...] = (
        lax.dot_general(x_ref[...], w_ref[...], (((1,), (1,)), ((), ())),
                        preferred_element_type=jnp.float32)
        + b_ref[...]
    )


def _fc1(x, w, b):
    n, d = x.shape
    blk = 2000
    return pl.pallas_call(
        _fc1_body,
        grid=(n // blk,),
        in_specs=[
            pl.BlockSpec((blk, d), lambda i: (i, 0)),
            pl.BlockSpec((d, d), lambda i: (0, 0)),
            pl.BlockSpec((1, d), lambda i: (0, 0)),
        ],
        out_specs=pl.BlockSpec((blk, d), lambda i: (i, 0)),
        out_shape=jax.ShapeDtypeStruct((n, d), jnp.float32),
    )(x, w, b.reshape(1, d))


def _make_seg_sum(n, d, e):
    ept = e // _NW
    k = 120
    nit = ept // k
    ktail = ept - nit * k
    npair = nit // 2
    nodd = nit - npair * 2
    assert ktail % 8 == 0
    rows_pt = (n // _NS) // 8 * 8
    rtail = n - rows_pt * _NS

    mesh = plsc.VectorSubcoreMesh(core_axis_name="c", subcore_axis_name="s")

    @functools.partial(
        pl.kernel,
        out_type=jax.ShapeDtypeStruct((_NC, n, d), jnp.float32),
        mesh=mesh,
        scratch_types=[
            pltpu.VMEM((ept,), jnp.int32),
            pltpu.VMEM((ept,), jnp.int32),
            pltpu.VMEM((k, d), jnp.float32),
            pltpu.VMEM((k, d), jnp.float32),
            pltpu.VMEM_SHARED((n, d), jnp.float32),
            pltpu.SemaphoreType.DMA,
            pltpu.SemaphoreType.DMA,
        ],
    )
    def seg_sum(h_hbm, src_hbm, dst_hbm, zeros_hbm, out_hbm,
                src_v, dst_v, rows_a, rows_b, acc_sh, sem_a, sem_b):
        cid = lax.axis_index("c")
        sid = lax.axis_index("s")
        wid = sid * _NC + cid
        pltpu.sync_copy(zeros_hbm.at[pl.ds(sid * rows_pt, rows_pt)],
                        acc_sh.at[pl.ds(sid * rows_pt, rows_pt)])
        if rtail:
            @pl.when(sid == _NS - 1)
            def _():
                pltpu.sync_copy(zeros_hbm.at[pl.ds(rows_pt * _NS, rtail)],
                                acc_sh.at[pl.ds(rows_pt * _NS, rtail)])
        plsc.subcore_barrier()

        base0 = wid * ept
        pltpu.sync_copy(src_hbm.at[pl.ds(base0, ept)], src_v)
        pltpu.sync_copy(dst_hbm.at[pl.ds(base0, ept)], dst_v)

        def gather(c, buf, sem):
            return pltpu.async_copy(
                h_hbm.at[src_v.at[pl.ds(c * k, k)]], buf, sem)

        def wait(buf, sem):
            pltpu.make_async_copy(h_hbm.at[pl.ds(0, k)], buf, sem).wait()

        def scatter(c, buf):
            pltpu.sync_copy(buf, acc_sh.at[dst_v.at[pl.ds(c * k, k)]],
                            add=True)

        gather(0, rows_a, sem_a)

        def body(j, carry):
            c0 = j * 2
            wait(rows_a, sem_a)
            gather(c0 + 1, rows_b, sem_b)
            scatter(c0, rows_a)
            wait(rows_b, sem_b)

            @pl.when(j < npair - 1 + nodd)
            def _():
                gather(c0 + 2, rows_a, sem_a)

            scatter(c0 + 1, rows_b)
            return carry

        lax.fori_loop(0, npair, body, 0)

        if nodd:
            wait(rows_a, sem_a)
            scatter(nit - 1, rows_a)
        if ktail:
            tb = nit * k
            pltpu.async_copy(
                h_hbm.at[src_v.at[pl.ds(tb, ktail)]],
                rows_b.at[pl.ds(0, ktail)], sem_b).wait()
            pltpu.sync_copy(rows_b.at[pl.ds(0, ktail)],
                            acc_sh.at[dst_v.at[pl.ds(tb, ktail)]], add=True)

        plsc.subcore_barrier()
        pltpu.sync_copy(acc_sh.at[pl.ds(sid * rows_pt, rows_pt)],
                        out_hbm.at[cid, pl.ds(sid * rows_pt, rows_pt)])
        if rtail:
            @pl.when(sid == _NS - 1)
            def _():
                pltpu.sync_copy(acc_sh.at[pl.ds(rows_pt * _NS, rtail)],
                                out_hbm.at[cid, pl.ds(rows_pt * _NS, rtail)])

    return seg_sum


def _mlp_body(scale_ref, h1_ref, a0_ref, a1_ref,
              w1_ref, b1_ref, w2_ref, b2_ref, bng_ref, bnb_ref,
              l1w_ref, l1b_ref, l2w_ref, l2b_ref, o_ref):
    u = scale_ref[0, 0] * h1_ref[...] + a0_ref[...] + a1_ref[...]
    dn = (((1,), (1,)), ((), ()))
    t = jnp.maximum(
        lax.dot_general(u, w1_ref[...], dn, preferred_element_type=jnp.float32)
        + b1_ref[...], 0.0)
    t = jnp.maximum(
        lax.dot_general(t, w2_ref[...], dn, preferred_element_type=jnp.float32)
        + b2_ref[...], 0.0)
    t = t * (bng_ref[...] * (1.0 / jnp.sqrt(1.0 + 1e-5))) + bnb_ref[...]
    t = jnp.maximum(
        lax.dot_general(t, l1w_ref[...], dn, preferred_element_type=jnp.float32)
        + l1b_ref[...], 0.0)
    o_ref[...] = (
        lax.dot_general(t, l2w_ref[...], dn, preferred_element_type=jnp.float32)
        + l2b_ref[...])


def _mlp(scale, h1, a0, a1, w1, b1, w2, b2, bn_g, bn_b, l1w, l1b, l2w, l2b):
    n, d = h1.shape
    h = w1.shape[0]
    blk = 2000
    full = lambda shape: pl.BlockSpec(shape, lambda i: tuple(0 for _ in shape))
    row = lambda width: pl.BlockSpec((blk, width), lambda i: (i, 0))
    return pl.pallas_call(
        _mlp_body,
        grid=(n // blk,),
        in_specs=[
            pl.BlockSpec(memory_space=pltpu.SMEM),
            row(d), row(d), row(d),
            full((h, d)), full((1, h)), full((h, h)), full((1, h)),
            full((1, h)), full((1, h)),
            full((h, h)), full((1, h)), full((h, h)), full((1, h)),
        ],
        out_specs=row(h),
        out_shape=jax.ShapeDtypeStruct((n, h), jnp.float32),
    )(scale, h1, a0, a1, w1, b1.reshape(1, h), w2, b2.reshape(1, h),
      bn_g.reshape(1, h), bn_b.reshape(1, h),
      l1w, l1b.reshape(1, h), l2w, l2b.reshape(1, h))


def _make_pair_gather(n, h, e, b):
    bpt = b // _NW
    m = 128
    nsub = bpt // m
    assert bpt % m == 0
    nv = h // 16

    mesh = plsc.VectorSubcoreMesh(core_axis_name="c", subcore_axis_name="s")

    @functools.partial(
        pl.kernel,
        out_type=jax.ShapeDtypeStruct((b, h), jnp.float32),
        mesh=mesh,
        scratch_types=[
            pltpu.VMEM((nsub, m), jnp.int32),
            pltpu.VMEM((nsub, m), jnp.int32),
            pltpu.VMEM((nsub, m), jnp.int32),
            pltpu.VMEM((m, h), jnp.float32),
            pltpu.VMEM((m, h), jnp.float32),
            pltpu.SemaphoreType.DMA,
            pltpu.SemaphoreType.DMA,
        ],
    )
    def pair(h_hbm, ei0_hbm, ei1_hbm, te_hbm, out_hbm,
             te_v, s_v, d_v, x1_v, x2_v, sem1, sem2):
        cid = lax.axis_index("c")
        sid = lax.axis_index("s")
        wid = sid * _NC + cid
        base = wid * bpt
        for j in range(nsub):
            pltpu.sync_copy(te_hbm.at[pl.ds(base + j * m, m)], te_v.at[j])
        for j in range(nsub):
            c1 = pltpu.async_copy(ei0_hbm.at[te_v.at[j]], s_v.at[j], sem1)
            c2 = pltpu.async_copy(ei1_hbm.at[te_v.at[j]], d_v.at[j], sem2)
            c1.wait()
            c2.wait()
        for j in range(nsub):
            c1 = pltpu.async_copy(h_hbm.at[s_v.at[j]], x1_v, sem1)
            c2 = pltpu.async_copy(h_hbm.at[d_v.at[j]], x2_v, sem2)
            c1.wait()
            c2.wait()

            def mul_row(r, carry):
                for c in range(nv):
                    sl = pl.ds(c * 16, 16)
                    x1_v[r, sl] = x1_v[r, sl] * x2_v[r, sl]
                return carry

            lax.fori_loop(0, m, mul_row, 0)
            pltpu.sync_copy(x1_v, out_hbm.at[pl.ds(base + j * m, m)])

    return pair


def _fc2_body(f_ref, w_ref, b_ref, o_ref):
    o_ref[...] = (
        lax.dot_general(f_ref[...], w_ref[...], (((1,), (1,)), ((), ())),
                        preferred_element_type=jnp.float32)
        + b_ref[...])


def _fc2(fused, w, bias):
    b, h = fused.shape
    c = w.shape[0]
    blk = 4096
    return pl.pallas_call(
        _fc2_body,
        grid=(b // blk,),
        in_specs=[
            pl.BlockSpec((blk, h), lambda i: (i, 0)),
            pl.BlockSpec((c, h), lambda i: (0, 0)),
            pl.BlockSpec((1, c), lambda i: (0, 0)),
        ],
        out_specs=pl.BlockSpec((blk, c), lambda i: (i, 0)),
        out_shape=jax.ShapeDtypeStruct((b, c), jnp.float32),
    )(fused, w, bias.reshape(1, c))


def kernel(x, edge_index, train_edge_id, fc1_w, fc1_b, eps, w1, b1, w2, b2,
           bn_g, bn_b, lin1_w, lin1_b, lin2_w, lin2_b, fc2_w, fc2_b):
    n, d = x.shape
    e = edge_index.shape[1]
    b = train_edge_id.shape[0]
    h = w1.shape[0]

    src = edge_index[0]
    dst = edge_index[1]

    h1 = _fc1(x, fc1_w, fc1_b)

    zeros = jnp.zeros((n, d), jnp.float32)
    aggs = _make_seg_sum(n, d, e)(h1, src, dst, zeros)

    scale = (1.0 + eps).reshape(1, 1)
    h5 = _mlp(scale, h1, aggs[0], aggs[1], w1, b1, w2, b2, bn_g, bn_b,
              lin1_w, lin1_b, lin2_w, lin2_b)

    fused = _make_pair_gather(n, h, e, b)(h5, src, dst, train_edge_id)

    return _fc2(fused, fc2_w, fc2_b)


# speedup vs baseline: 7.8818x; 1.0652x over previous
import functools

import jax
import jax.numpy as jnp
from jax import lax
from jax.experimental import pallas as pl
from jax.experimental.pallas import tpu as pltpu
from jax.experimental.pallas import tpu_sc as plsc

_NC = 2
_NS = 16
_NW = _NC * _NS


def _fc1_body(x_ref, w_ref, b_ref, o_ref):
    o_ref[...] = (
        lax.dot_general(x_ref[...], w_ref[...], (((1,), (1,)), ((), ())),
                        preferred_element_type=jnp.float32)
        + b_ref[...]
    )


def _fc1(x, w, b):
    n, d = x.shape
    blk = 2000
    return pl.pallas_call(
        _fc1_body,
        grid=(n // blk,),
        in_specs=[
            pl.BlockSpec((blk, d), lambda i: (i, 0)),
            pl.BlockSpec((d, d), lambda i: (0, 0)),
            pl.BlockSpec((1, d), lambda i: (0, 0)),
        ],
        out_specs=pl.BlockSpec((blk, d), lambda i: (i, 0)),
        out_shape=jax.ShapeDtypeStruct((n, d), jnp.float32),
    )(x, w, b.reshape(1, d))


def _make_seg_sum(n, d, e):
    ept = e // _NW
    k = 120
    nit = ept // k
    ktail = ept - nit * k
    npair = nit // 2
    nodd = nit - npair * 2
    assert ktail % 8 == 0
    rows_pt = (n // _NS) // 8 * 8
    rtail = n - rows_pt * _NS

    mesh = plsc.VectorSubcoreMesh(core_axis_name="c", subcore_axis_name="s")

    @functools.partial(
        pl.kernel,
        out_type=jax.ShapeDtypeStruct((_NC, n, d), jnp.float32),
        mesh=mesh,
        scratch_types=[
            pltpu.VMEM((ept,), jnp.int32),
            pltpu.VMEM((ept,), jnp.int32),
            pltpu.VMEM((k, d), jnp.float32),
            pltpu.VMEM((k, d), jnp.float32),
            pltpu.VMEM_SHARED((n, d), jnp.float32),
            pltpu.SemaphoreType.DMA,
            pltpu.SemaphoreType.DMA,
        ],
    )
    def seg_sum(h_hbm, src_hbm, dst_hbm, zeros_hbm, out_hbm,
                src_v, dst_v, rows_a, rows_b, acc_sh, sem_a, sem_b):
        cid = lax.axis_index("c")
        sid = lax.axis_index("s")
        wid = sid * _NC + cid
        pltpu.sync_copy(zeros_hbm.at[pl.ds(sid * rows_pt, rows_pt)],
                        acc_sh.at[pl.ds(sid * rows_pt, rows_pt)])
        if rtail:
            @pl.when(sid == _NS - 1)
            def _():
                pltpu.sync_copy(zeros_hbm.at[pl.ds(rows_pt * _NS, rtail)],
                                acc_sh.at[pl.ds(rows_pt * _NS, rtail)])
        plsc.subcore_barrier()

        base0 = wid * ept
        pltpu.sync_copy(src_hbm.at[pl.ds(base0, ept)], src_v)
        pltpu.sync_copy(dst_hbm.at[pl.ds(base0, ept)], dst_v)

        def gather(c, buf, sem):
            return pltpu.async_copy(
                h_hbm.at[src_v.at[pl.ds(c * k, k)]], buf, sem)

        def wait(buf, sem):
            pltpu.make_async_copy(h_hbm.at[pl.ds(0, k)], buf, sem).wait()

        def scatter(c, buf):
            pltpu.sync_copy(buf, acc_sh.at[dst_v.at[pl.ds(c * k, k)]],
                            add=True)

        gather(0, rows_a, sem_a)

        def body(j, carry):
            c0 = j * 2
            wait(rows_a, sem_a)
            gather(c0 + 1, rows_b, sem_b)
            scatter(c0, rows_a)
            wait(rows_b, sem_b)

            @pl.when(j < npair - 1 + nodd)
            def _():
                gather(c0 + 2, rows_a, sem_a)

            scatter(c0 + 1, rows_b)
            return carry

        lax.fori_loop(0, npair, body, 0)

        if nodd:
            wait(rows_a, sem_a)
            scatter(nit - 1, rows_a)
        if ktail:
            tb = nit * k
            pltpu.async_copy(
                h_hbm.at[src_v.at[pl.ds(tb, ktail)]],
                rows_b.at[pl.ds(0, ktail)], sem_b).wait()
            pltpu.sync_copy(rows_b.at[pl.ds(0, ktail)],
                            acc_sh.at[dst_v.at[pl.ds(tb, ktail)]], add=True)

        plsc.subcore_barrier()
        pltpu.sync_copy(acc_sh.at[pl.ds(sid * rows_pt, rows_pt)],
                        out_hbm.at[cid, pl.ds(sid * rows_pt, rows_pt)])
        if rtail:
            @pl.when(sid == _NS - 1)
            def _():
                pltpu.sync_copy(acc_sh.at[pl.ds(rows_pt * _NS, rtail)],
                                out_hbm.at[cid, pl.ds(rows_pt * _NS, rtail)])

    return seg_sum


def _mlp_body(scale_ref, h1_ref, a0_ref, a1_ref,
              w1_ref, b1_ref, w2_ref, b2_ref, bng_ref, bnb_ref,
              l1w_ref, l1b_ref, l2w_ref, l2b_ref, o_ref):
    u = scale_ref[0, 0] * h1_ref[...] + a0_ref[0] + a1_ref[0]
    dn = (((1,), (1,)), ((), ()))
    t = jnp.maximum(
        lax.dot_general(u, w1_ref[...], dn, preferred_element_type=jnp.float32)
        + b1_ref[...], 0.0)
    t = jnp.maximum(
        lax.dot_general(t, w2_ref[...], dn, preferred_element_type=jnp.float32)
        + b2_ref[...], 0.0)
    t = t * (bng_ref[...] * (1.0 / jnp.sqrt(1.0 + 1e-5))) + bnb_ref[...]
    t = jnp.maximum(
        lax.dot_general(t, l1w_ref[...], dn, preferred_element_type=jnp.float32)
        + l1b_ref[...], 0.0)
    o_ref[...] = (
        lax.dot_general(t, l2w_ref[...], dn, preferred_element_type=jnp.float32)
        + l2b_ref[...])


def _mlp(scale, h1, aggs, w1, b1, w2, b2, bn_g, bn_b, l1w, l1b, l2w, l2b):
    n, d = h1.shape
    h = w1.shape[0]
    blk = 2000
    full = lambda shape: pl.BlockSpec(shape, lambda i: tuple(0 for _ in shape))
    row = lambda width: pl.BlockSpec((blk, width), lambda i: (i, 0))
    return pl.pallas_call(
        _mlp_body,
        grid=(n // blk,),
        in_specs=[
            pl.BlockSpec(memory_space=pltpu.SMEM),
            row(d),
            pl.BlockSpec((1, blk, d), lambda i: (0, i, 0)),
            pl.BlockSpec((1, blk, d), lambda i: (1, i, 0)),
            full((h, d)), full((1, h)), full((h, h)), full((1, h)),
            full((1, h)), full((1, h)),
            full((h, h)), full((1, h)), full((h, h)), full((1, h)),
        ],
        out_specs=row(h),
        out_shape=jax.ShapeDtypeStruct((n, h), jnp.float32),
    )(scale, h1, aggs, aggs, w1, b1.reshape(1, h), w2, b2.reshape(1, h),
      bn_g.reshape(1, h), bn_b.reshape(1, h),
      l1w, l1b.reshape(1, h), l2w, l2b.reshape(1, h))


def _make_pair_gather(n, h, e, b):
    bpt = b // _NW
    m = 128
    nsub = bpt // m
    assert bpt % m == 0
    nv = h // 16

    mesh = plsc.VectorSubcoreMesh(core_axis_name="c", subcore_axis_name="s")

    @functools.partial(
        pl.kernel,
        out_type=jax.ShapeDtypeStruct((b, h), jnp.float32),
        mesh=mesh,
        scratch_types=[
            pltpu.VMEM((nsub, m), jnp.int32),
            pltpu.VMEM((nsub, m), jnp.int32),
            pltpu.VMEM((nsub, m), jnp.int32),
            pltpu.VMEM((m, h), jnp.float32),
            pltpu.VMEM((m, h), jnp.float32),
            pltpu.SemaphoreType.DMA,
            pltpu.SemaphoreType.DMA,
        ],
    )
    def pair(h_hbm, ei0_hbm, ei1_hbm, te_hbm, out_hbm,
             te_v, s_v, d_v, x1_v, x2_v, sem1, sem2):
        cid = lax.axis_index("c")
        sid = lax.axis_index("s")
        wid = sid * _NC + cid
        base = wid * bpt
        for j in range(nsub):
            pltpu.sync_copy(te_hbm.at[pl.ds(base + j * m, m)], te_v.at[j])
        for j in range(nsub):
            c1 = pltpu.async_copy(ei0_hbm.at[te_v.at[j]], s_v.at[j], sem1)
            c2 = pltpu.async_copy(ei1_hbm.at[te_v.at[j]], d_v.at[j], sem2)
            c1.wait()
            c2.wait()
        for j in range(nsub):
            c1 = pltpu.async_copy(h_hbm.at[s_v.at[j]], x1_v, sem1)
            c2 = pltpu.async_copy(h_hbm.at[d_v.at[j]], x2_v, sem2)
            c1.wait()
            c2.wait()

            def mul_row(r, carry):
                for c in range(nv):
                    sl = pl.ds(c * 16, 16)
                    x1_v[r, sl] = x1_v[r, sl] * x2_v[r, sl]
                return carry

            lax.fori_loop(0, m, mul_row, 0)
            pltpu.sync_copy(x1_v, out_hbm.at[pl.ds(base + j * m, m)])

    return pair


def _fc2_body(f_ref, w_ref, b_ref, o_ref):
    o_ref[...] = (
        lax.dot_general(w_ref[...], f_ref[...], (((1,), (1,)), ((), ())),
                        preferred_element_type=jnp.float32)
        + b_ref[...])


def _fc2(fused, w, bias):
    b, h = fused.shape
    c = w.shape[0]
    blk = 4096
    out_t = pl.pallas_call(
        _fc2_body,
        grid=(b // blk,),
        in_specs=[
            pl.BlockSpec((blk, h), lambda i: (i, 0)),
            pl.BlockSpec((c, h), lambda i: (0, 0)),
            pl.BlockSpec((c, 1), lambda i: (0, 0)),
        ],
        out_specs=pl.BlockSpec((c, blk), lambda i: (0, i)),
        out_shape=jax.ShapeDtypeStruct((c, b), jnp.float32),
    )(fused, w, bias.reshape(c, 1))
    return out_t.T


def kernel(x, edge_index, train_edge_id, fc1_w, fc1_b, eps, w1, b1, w2, b2,
           bn_g, bn_b, lin1_w, lin1_b, lin2_w, lin2_b, fc2_w, fc2_b):
    n, d = x.shape
    e = edge_index.shape[1]
    b = train_edge_id.shape[0]
    h = w1.shape[0]

    src = edge_index[0]
    dst = edge_index[1]

    h1 = _fc1(x, fc1_w, fc1_b)

    zeros = jnp.zeros((n, d), jnp.float32)
    aggs = _make_seg_sum(n, d, e)(h1, src, dst, zeros)

    scale = (1.0 + eps).reshape(1, 1)
    h5 = _mlp(scale, h1, aggs, w1, b1, w2, b2, bn_g, bn_b,
              lin1_w, lin1_b, lin2_w, lin2_b)

    fused = _make_pair_gather(n, h, e, b)(h5, src, dst, train_edge_id)

    return _fc2(fused, fc2_w, fc2_b)
